# Initial kernel scaffold; baseline (speedup 1.0000x reference)
import jax
import jax.numpy as jnp
from jax.experimental import pallas as pl


def kernel(in_feat, table):
    raise NotImplementedError("write your pallas kernel here")


# speedup vs baseline: 1.1034x; 1.1034x over previous
import functools

import jax
import jax.numpy as jnp
from jax import lax
from jax.experimental import pallas as pl
from jax.experimental.pallas import tpu as pltpu
from jax.experimental.pallas import tpu_sc as plsc


def _make_gather(n_rows: int, d: int, chunk: int):
    info = plsc.get_sparse_core_info()
    nw = info.num_cores * info.num_subcores
    assert n_rows % nw == 0
    per_w = n_rows // nw
    assert per_w % chunk == 0
    n_chunks = per_w // chunk

    mesh = plsc.VectorSubcoreMesh(core_axis_name="c", subcore_axis_name="s")

    @functools.partial(
        pl.kernel,
        out_type=jax.ShapeDtypeStruct((n_rows, d), jnp.float32),
        mesh=mesh,
        scratch_types=[
            pltpu.VMEM((chunk,), jnp.int32),
            pltpu.VMEM((chunk, d), jnp.float32),
            pltpu.SemaphoreType.DMA,
        ],
        compiler_params=pltpu.CompilerParams(use_tc_tiling_on_sc=False),
    )
    def gather_kernel(idx_hbm, table_hbm, out_hbm, idx_v, rows_v, sem):
        wid = lax.axis_index("s") * info.num_cores + lax.axis_index("c")
        base = wid * per_w

        def body(i, carry):
            off = base + i * chunk
            pltpu.sync_copy(idx_hbm.at[pl.ds(off, chunk)], idx_v)
            pltpu.async_copy(table_hbm.at[idx_v], rows_v, sem).wait()
            pltpu.sync_copy(rows_v, out_hbm.at[pl.ds(off, chunk)])
            return carry

        lax.fori_loop(0, n_chunks, body, 0)

    return gather_kernel


def kernel(in_feat, table):
    b, h = in_feat.shape
    v, d = table.shape
    n = b * h
    idx = in_feat.reshape(n).astype(jnp.int32)
    out = _make_gather(n, d, chunk=1600)(idx, table)
    return out.reshape(b, h, d)


# speedup vs baseline: 1.2792x; 1.1594x over previous
import functools

import jax
import jax.numpy as jnp
from jax import lax
from jax.experimental import pallas as pl
from jax.experimental.pallas import tpu as pltpu
from jax.experimental.pallas import tpu_sc as plsc


def _make_gather(n_rows: int, d: int, chunk: int, nbuf: int):
    info = plsc.get_sparse_core_info()
    nw = info.num_cores * info.num_subcores
    assert n_rows % nw == 0
    per_w = n_rows // nw
    assert per_w % chunk == 0
    m = per_w // chunk
    assert m % nbuf == 0 and m >= nbuf
    n_chunks_total = n_rows // chunk

    mesh = plsc.VectorSubcoreMesh(core_axis_name="c", subcore_axis_name="s")

    @functools.partial(
        pl.kernel,
        out_type=jax.ShapeDtypeStruct((n_chunks_total, chunk, d), jnp.float32),
        mesh=mesh,
        scratch_types=[
            pltpu.VMEM((m, chunk), jnp.int32),
            pltpu.VMEM((nbuf, chunk, d), jnp.float32),
        ]
        + [pltpu.SemaphoreType.DMA] * (2 * nbuf),
        compiler_params=pltpu.CompilerParams(use_tc_tiling_on_sc=False),
    )
    def gather_kernel(idx_hbm, table_hbm, out_hbm, idx_v, rows_v, *sems):
        gsem = sems[:nbuf]
        ssem = sems[nbuf:]
        wid = lax.axis_index("s") * info.num_cores + lax.axis_index("c")
        row0 = wid * m

        pltpu.sync_copy(idx_hbm.at[pl.ds(row0, m)], idx_v)

        def gather_copy(ci, b):
            return pltpu.make_async_copy(
                table_hbm.at[idx_v.at[ci]], rows_v.at[b], gsem[b]
            )

        def store_copy(ci, b):
            return pltpu.make_async_copy(
                rows_v.at[b], out_hbm.at[row0 + ci], ssem[b]
            )

        for b in range(nbuf - 1):
            gather_copy(b, b).start()

        def outer_body(o, carry):
            for b in range(nbuf):
                i = o * nbuf + b
                j = i + nbuf - 1
                bj = (b + nbuf - 1) % nbuf

                if b == 0:
                    @pl.when(o > 0)
                    def _():
                        store_copy(i - 1, bj).wait()
                else:
                    store_copy(i - 1, bj).wait()

                if b == 0:
                    gather_copy(j, bj).start()
                else:
                    @pl.when(j < m)
                    def _():
                        gather_copy(j, bj).start()

                gather_copy(i, b).wait()
                store_copy(i, b).start()
            return carry

        lax.fori_loop(0, m // nbuf, outer_body, 0)
        store_copy(m - 1, (m - 1) % nbuf).wait()

    return gather_kernel


def kernel(in_feat, table):
    b, h = in_feat.shape
    v, d = table.shape
    n = b * h
    chunk = 640
    idx = in_feat.reshape(n // chunk, chunk).astype(jnp.int32)
    out = _make_gather(n, d, chunk=chunk, nbuf=4)(idx, table)
    return out.reshape(b, h, d)
